# Initial kernel scaffold; baseline (speedup 1.0000x reference)
import jax
import jax.numpy as jnp
from jax.experimental import pallas as pl


def kernel(t, table):
    raise NotImplementedError("write your pallas kernel here")


# speedup vs baseline: 4.8341x; 4.8341x over previous
import functools

import jax
import jax.numpy as jnp
from jax import lax
from jax.experimental import pallas as pl
from jax.experimental.pallas import tpu as pltpu
from jax.experimental.pallas import tpu_sc as plsc

_INFO = plsc.get_sparse_core_info()
_NC = _INFO.num_cores
_NS = _INFO.num_subcores
_NW = _NC * _NS

_CHUNK = 128
_NBUF = 2


def _make_gather(vocab, dim, batch):
    assert batch % (_NW * _CHUNK) == 0
    b_per_w = batch // _NW
    n_chunks = b_per_w // _CHUNK
    mesh = plsc.VectorSubcoreMesh(core_axis_name="c", subcore_axis_name="s")

    @functools.partial(
        pl.kernel,
        mesh=mesh,
        compiler_params=pltpu.CompilerParams(use_tc_tiling_on_sc=False),
        out_type=jax.ShapeDtypeStruct((batch, dim), jnp.float32),
        scratch_types=[
            pltpu.VMEM((n_chunks, _CHUNK), jnp.int32),
            pltpu.VMEM((_NBUF, _CHUNK, dim), jnp.float32),
            pltpu.SemaphoreType.DMA,
            pltpu.SemaphoreType.DMA,
        ],
    )
    def gather_kernel(table_hbm, idx_hbm, out_hbm, idx_v, rows_v, sem0, sem1):
        wid = lax.axis_index("s") * _NC + lax.axis_index("c")
        base = wid * b_per_w
        sems = (sem0, sem1)

        pltpu.sync_copy(idx_hbm.at[wid], idx_v)

        def start(j, b):
            pltpu.async_copy(table_hbm.at[idx_v.at[j]], rows_v.at[b], sems[b])

        def wait(j, b):
            pltpu.make_async_copy(
                table_hbm.at[idx_v.at[j]], rows_v.at[b], sems[b]
            ).wait()

        for b in range(_NBUF):
            start(b, b)

        def loop_body(g, carry):
            for b in range(_NBUF):
                j = g * _NBUF + b
                wait(j, b)
                pltpu.sync_copy(
                    rows_v.at[b], out_hbm.at[pl.ds(base + j * _CHUNK, _CHUNK)]
                )
                nj = j + _NBUF

                @pl.when(nj < n_chunks)
                def _():
                    start(nj, b)

            return carry

        lax.fori_loop(0, n_chunks // _NBUF, loop_body, 0)

    return gather_kernel


def kernel(t, table):
    vocab, dim = table.shape
    batch = t.size
    idx = t.reshape(_NW, batch // (_NW * _CHUNK), _CHUNK).astype(jnp.int32)
    out = _make_gather(vocab, dim, batch)(table, idx)
    return out.reshape(t.shape + (dim,))


# speedup vs baseline: 4.9380x; 1.0215x over previous
import functools

import jax
import jax.numpy as jnp
from jax import lax
from jax.experimental import pallas as pl
from jax.experimental.pallas import tpu as pltpu
from jax.experimental.pallas import tpu_sc as plsc

_INFO = plsc.get_sparse_core_info()
_NC = _INFO.num_cores
_NS = _INFO.num_subcores
_NW = _NC * _NS

_IDXW = 128
_SUB = 2
_CHUNK = _IDXW * _SUB
_NBUF = 4
_LOOK = 2


def _make_gather(vocab, dim, batch):
    assert batch % (_NW * _CHUNK) == 0
    b_per_w = batch // _NW
    n_chunks = b_per_w // _CHUNK
    n_groups = n_chunks // _NBUF
    assert n_chunks % _NBUF == 0 and n_groups >= 2
    mesh = plsc.VectorSubcoreMesh(core_axis_name="c", subcore_axis_name="s")

    @functools.partial(
        pl.kernel,
        mesh=mesh,
        compiler_params=pltpu.CompilerParams(use_tc_tiling_on_sc=False),
        out_type=jax.ShapeDtypeStruct((batch, dim), jnp.float32),
        scratch_types=[
            pltpu.VMEM((b_per_w // _IDXW, _IDXW), jnp.int32),
            pltpu.VMEM((_NBUF, _CHUNK, dim), jnp.float32),
            [pltpu.SemaphoreType.DMA] * _NBUF,
            [pltpu.SemaphoreType.DMA] * _NBUF,
        ],
    )
    def gather_kernel(table_hbm, idx_hbm, out_hbm, idx_v, rows_v, gsem, ssem):
        wid = lax.axis_index("s") * _NC + lax.axis_index("c")
        base = wid * b_per_w

        pltpu.sync_copy(idx_hbm.at[wid], idx_v)

        def gather_descr(j, b, k):
            return pltpu.make_async_copy(
                table_hbm.at[idx_v.at[j * _SUB + k]],
                rows_v.at[b].at[pl.ds(k * _IDXW, _IDXW)],
                gsem[b],
            )

        def store_descr(j, b):
            return pltpu.make_async_copy(
                rows_v.at[b],
                out_hbm.at[pl.ds(base + j * _CHUNK, _CHUNK)],
                ssem[b],
            )

        def start_gather(j, b):
            for k in range(_SUB):
                gather_descr(j, b, k).start()

        def wait_gather(j, b):
            for k in range(_SUB):
                gather_descr(j, b, k).wait()

        def group(g, first=False, last=False):
            for b in range(_NBUF):
                j = g * _NBUF + b
                wait_gather(j, b)
                store_descr(j, b).start()
                if last and b >= _NBUF - _LOOK:
                    continue
                nj = j + _LOOK
                nb = (b + _LOOK) % _NBUF
                if not (first and b < _LOOK):
                    store_descr(nj - _NBUF, nb).wait()
                start_gather(nj, nb)

        for j in range(_LOOK):
            start_gather(j, j % _NBUF)
        group(0, first=True)
        lax.fori_loop(1, n_groups - 1, lambda g, c: (group(g), c)[1], 0)
        group(n_groups - 1, last=True)
        for b in range(_NBUF):
            store_descr(n_chunks - _NBUF + b, b).wait()

    return gather_kernel


def kernel(t, table):
    vocab, dim = table.shape
    batch = t.size
    idx = t.reshape(_NW, batch // (_NW * _IDXW), _IDXW).astype(jnp.int32)
    out = _make_gather(vocab, dim, batch)(table, idx)
    return out.reshape(t.shape + (dim,))


# speedup vs baseline: 4.9499x; 1.0024x over previous
import functools

import jax
import jax.numpy as jnp
from jax import lax
from jax.experimental import pallas as pl
from jax.experimental.pallas import tpu as pltpu
from jax.experimental.pallas import tpu_sc as plsc

_INFO = plsc.get_sparse_core_info()
_NC = _INFO.num_cores
_NS = _INFO.num_subcores
_NW = _NC * _NS

_SUB = 2
_NBUF = 4
_LOOK = 2


def _make_gather(vocab, dim, n_seq, seq_len):
    assert n_seq % _NW == 0 and seq_len % _SUB == 0
    idxw = seq_len // _SUB
    assert idxw <= 128
    s_per_w = n_seq // _NW
    n_groups = s_per_w // _NBUF
    assert s_per_w % _NBUF == 0 and n_groups >= 2
    mesh = plsc.VectorSubcoreMesh(core_axis_name="c", subcore_axis_name="s")

    @functools.partial(
        pl.kernel,
        mesh=mesh,
        compiler_params=pltpu.CompilerParams(use_tc_tiling_on_sc=False),
        out_type=jax.ShapeDtypeStruct((n_seq, seq_len, dim), jnp.float32),
        scratch_types=[
            pltpu.VMEM((s_per_w * _SUB, idxw), jnp.int32),
            pltpu.VMEM((_NBUF, seq_len, dim), jnp.float32),
            [pltpu.SemaphoreType.DMA] * _NBUF,
            [pltpu.SemaphoreType.DMA] * _NBUF,
        ],
    )
    def gather_kernel(table_hbm, idx_hbm, out_hbm, idx_v, rows_v, gsem, ssem):
        wid = lax.axis_index("s") * _NC + lax.axis_index("c")
        base = wid * s_per_w

        pltpu.sync_copy(idx_hbm.at[wid], idx_v)

        def gather_descr(j, b, k):
            return pltpu.make_async_copy(
                table_hbm.at[idx_v.at[j * _SUB + k]],
                rows_v.at[b].at[pl.ds(k * idxw, idxw)],
                gsem[b],
            )

        def store_descr(j, b):
            return pltpu.make_async_copy(
                rows_v.at[b],
                out_hbm.at[base + j],
                ssem[b],
            )

        def start_gather(j, b):
            for k in range(_SUB):
                gather_descr(j, b, k).start()

        def wait_gather(j, b):
            for k in range(_SUB):
                gather_descr(j, b, k).wait()

        def group(g, first=False, last=False):
            for b in range(_NBUF):
                j = g * _NBUF + b
                wait_gather(j, b)
                store_descr(j, b).start()
                if last and b >= _NBUF - _LOOK:
                    continue
                nj = j + _LOOK
                nb = (b + _LOOK) % _NBUF
                if not (first and b < _LOOK):
                    store_descr(nj - _NBUF, nb).wait()
                start_gather(nj, nb)

        for j in range(_LOOK):
            start_gather(j, j % _NBUF)
        group(0, first=True)
        lax.fori_loop(1, n_groups - 1, lambda g, c: (group(g), c)[1], 0)
        group(n_groups - 1, last=True)
        for b in range(_NBUF):
            store_descr(s_per_w - _NBUF + b, b).wait()

    return gather_kernel


def kernel(t, table):
    vocab, dim = table.shape
    n_seq, seq_len = t.shape
    idx = t.reshape(_NW, (n_seq // _NW) * _SUB, seq_len // _SUB).astype(
        jnp.int32
    )
    return _make_gather(vocab, dim, n_seq, seq_len)(table, idx)
